# Initial kernel scaffold; baseline (speedup 1.0000x reference)
import jax
import jax.numpy as jnp
from jax.experimental import pallas as pl


def kernel(x, edge_index, batch, W1, b1, W2, b2, W3, b3):
    raise NotImplementedError("write your pallas kernel here")


# speedup vs baseline: 9.6089x; 9.6089x over previous
import functools

import jax
import jax.numpy as jnp
from jax import lax
from jax.experimental import pallas as pl
from jax.experimental.pallas import tpu as pltpu
from jax.experimental.pallas import tpu_sc as plsc

N = 10000
NPAD = 10240
E = 320000
EP = 331776
G = 64
NC = 2
NS = 16
ROWS_PER_TILE = NPAD // NS
K = 128
BN = 2048
NB = NPAD // BN


def _sc_pass(table, src_idx, dst_idx, zeros, width, feat_split):
    if feat_split:
        edges_per_tile = EP // NS
    else:
        edges_per_tile = EP // (NC * NS)
    nchunks = edges_per_tile // K

    mesh = plsc.VectorSubcoreMesh(core_axis_name="c", subcore_axis_name="s")

    @functools.partial(
        pl.kernel,
        out_type=jax.ShapeDtypeStruct((NC * NPAD, width), jnp.float32),
        mesh=mesh,
        scratch_types=[
            pltpu.VMEM((K,), jnp.int32),
            pltpu.VMEM((K,), jnp.int32),
            pltpu.VMEM((K, width), jnp.float32),
            pltpu.VMEM_SHARED((NPAD, width), jnp.float32),
            pltpu.SemaphoreType.DMA,
        ],
        compiler_params=pltpu.CompilerParams(use_tc_tiling_on_sc=False),
    )
    def k(table_h, src_h, dst_h, zero_h, out_h, src_v, dst_v, rows_v, agg_sh,
          sem):
        c = lax.axis_index("c")
        s = lax.axis_index("s")
        r0 = s * ROWS_PER_TILE
        pltpu.sync_copy(zero_h.at[pl.ds(r0, ROWS_PER_TILE)],
                        agg_sh.at[pl.ds(r0, ROWS_PER_TILE)])
        plsc.subcore_barrier()

        if feat_split:
            sbase = c * EP + s * edges_per_tile
            dbase = s * edges_per_tile
        else:
            sbase = (c * NS + s) * edges_per_tile
            dbase = sbase

        def body(i, carry):
            off = i * K
            pltpu.sync_copy(src_h.at[pl.ds(sbase + off, K)], src_v)
            pltpu.sync_copy(dst_h.at[pl.ds(dbase + off, K)], dst_v)
            pltpu.async_copy(table_h.at[src_v], rows_v, sem).wait()
            pltpu.sync_copy(rows_v, agg_sh.at[dst_v], add=True)
            return carry

        lax.fori_loop(0, nchunks, body, 0)
        plsc.subcore_barrier()
        pltpu.sync_copy(agg_sh.at[pl.ds(r0, ROWS_PER_TILE)],
                        out_h.at[pl.ds(c * NPAD + r0, ROWS_PER_TILE)])

    return k(table, src_idx, dst_idx, zeros)


def _t1(xp, W1, cnt0, cnt1):
    def body(x_ref, w_ref, c0_ref, c1_ref, h_ref, dinv_ref):
        deg = c0_ref[...][:, :1] + c1_ref[...][:, :1]
        dinv = jnp.where(deg > 0.0, lax.rsqrt(deg), 0.0)
        dinv_ref[...] = dinv
        h = jnp.dot(x_ref[...], w_ref[...],
                    preferred_element_type=jnp.float32) * dinv
        h_ref[0] = h[:, :64]
        h_ref[1] = h[:, 64:]

    return pl.pallas_call(
        body,
        grid=(NB,),
        in_specs=[
            pl.BlockSpec((BN, 128), lambda i: (i, 0)),
            pl.BlockSpec((128, 128), lambda i: (0, 0)),
            pl.BlockSpec((BN, 16), lambda i: (i, 0)),
            pl.BlockSpec((BN, 16), lambda i: (i, 0)),
        ],
        out_specs=[
            pl.BlockSpec((NC, BN, 64), lambda i: (0, i, 0)),
            pl.BlockSpec((BN, 1), lambda i: (i, 0)),
        ],
        out_shape=[
            jax.ShapeDtypeStruct((NC, NPAD, 64), jnp.float32),
            jax.ShapeDtypeStruct((NPAD, 1), jnp.float32),
        ],
    )(xp, W1, cnt0, cnt1)


def _mid(a0, a1, dinv, b, W, split_out):
    outw = W.shape[1]

    def body(a0_ref, a1_ref, dinv_ref, b_ref, w_ref, out_ref):
        agg = jnp.concatenate([a0_ref[...], a1_ref[...]], axis=1)
        dinv = dinv_ref[...]
        z = jnp.maximum(agg * dinv + b_ref[...], 0.0)
        r = jnp.dot(z, w_ref[...], preferred_element_type=jnp.float32) * dinv
        if split_out:
            out_ref[0] = r[:, :64]
            out_ref[1] = r[:, 64:]
        else:
            out_ref[...] = r

    if split_out:
        out_spec = pl.BlockSpec((NC, BN, 64), lambda i: (0, i, 0))
        out_shape = jax.ShapeDtypeStruct((NC, NPAD, 64), jnp.float32)
    else:
        out_spec = pl.BlockSpec((BN, outw), lambda i: (i, 0))
        out_shape = jax.ShapeDtypeStruct((NPAD, outw), jnp.float32)

    return pl.pallas_call(
        body,
        grid=(NB,),
        in_specs=[
            pl.BlockSpec((BN, 64), lambda i: (i, 0)),
            pl.BlockSpec((BN, 64), lambda i: (i, 0)),
            pl.BlockSpec((BN, 1), lambda i: (i, 0)),
            pl.BlockSpec((1, 128), lambda i: (0, 0)),
            pl.BlockSpec((128, outw), lambda i: (0, 0)),
        ],
        out_specs=out_spec,
        out_shape=out_shape,
    )(a0, a1, dinv, b, W)


def _t4(g0, g1, dinv, b3p, batr):
    def body(g0_ref, g1_ref, dinv_ref, b_ref, bat_ref, out_ref):
        i = pl.program_id(0)

        @pl.when(i == 0)
        def _init():
            out_ref[...] = jnp.zeros_like(out_ref)

        p = (g0_ref[...] + g1_ref[...]) * dinv_ref[...] + b_ref[...]
        col = lax.broadcasted_iota(jnp.int32, (BN, 16), 1)
        p_aug = jnp.where(col == 15, 1.0, p)
        gids = lax.broadcasted_iota(jnp.int32, (G, BN), 0)
        mask = (bat_ref[...] == gids).astype(jnp.float32)
        out_ref[...] += jnp.dot(mask, p_aug,
                                preferred_element_type=jnp.float32)

        @pl.when(i == NB - 1)
        def _final():
            sums = out_ref[...]
            cnt = jnp.maximum(sums[:, 15:16], 1.0)
            m = sums / cnt
            ccol = lax.broadcasted_iota(jnp.int32, (G, 16), 1)
            logits = jnp.where(ccol < 10, m, -1e30)
            zz = logits - jnp.max(logits, axis=1, keepdims=True)
            ez = jnp.exp(zz)
            out_ref[...] = ez / jnp.sum(ez, axis=1, keepdims=True)

    return pl.pallas_call(
        body,
        grid=(NB,),
        in_specs=[
            pl.BlockSpec((BN, 16), lambda i: (i, 0)),
            pl.BlockSpec((BN, 16), lambda i: (i, 0)),
            pl.BlockSpec((BN, 1), lambda i: (i, 0)),
            pl.BlockSpec((1, 16), lambda i: (0, 0)),
            pl.BlockSpec((1, BN), lambda i: (0, i)),
        ],
        out_specs=pl.BlockSpec((G, 16), lambda i: (0, 0)),
        out_shape=jax.ShapeDtypeStruct((G, 16), jnp.float32),
    )(g0, g1, dinv, b3p, batr)


def kernel(x, edge_index, batch, W1, b1, W2, b2, W3, b3):
    src = edge_index[0]
    dst = edge_index[1]
    loop = jnp.arange(N, dtype=jnp.int32)
    padn = jnp.full((EP - E - N,), N, dtype=jnp.int32)
    srcf = jnp.concatenate([src, loop, padn])
    dstf = jnp.concatenate([dst, loop, padn])
    src2 = jnp.concatenate([srcf, srcf + NPAD])

    xp = jnp.zeros((NPAD, 128), jnp.float32).at[:N].set(x)
    batr = jnp.full((NPAD,), G, jnp.int32).at[:N].set(batch).reshape(1, NPAD)
    W3p = jnp.zeros((128, 16), jnp.float32).at[:, :10].set(W3)
    b1r = b1.reshape(1, 128)
    b2r = b2.reshape(1, 128)
    b3p = jnp.zeros((1, 16), jnp.float32).at[0, :10].set(b3)
    ones16 = jnp.ones((NPAD, 16), jnp.float32)
    z16 = jnp.zeros((NPAD, 16), jnp.float32)
    z64 = jnp.zeros((NPAD, 64), jnp.float32)

    cnt = _sc_pass(ones16, srcf, dstf, z16, 16, False)
    h1, dinv = _t1(xp, W1, cnt[:NPAD], cnt[NPAD:])
    agg1 = _sc_pass(h1.reshape(NC * NPAD, 64), src2, dstf, z64, 64, True)
    h2 = _mid(agg1[:NPAD], agg1[NPAD:], dinv, b1r, W2, True)
    agg2 = _sc_pass(h2.reshape(NC * NPAD, 64), src2, dstf, z64, 64, True)
    h3 = _mid(agg2[:NPAD], agg2[NPAD:], dinv, b2r, W3p, False)
    agg3 = _sc_pass(h3, srcf, dstf, z16, 16, False)
    out = _t4(agg3[:NPAD], agg3[NPAD:], dinv, b3p, batr)
    return out[:, :10]


# speedup vs baseline: 19.1630x; 1.9943x over previous
import functools

import jax
import jax.numpy as jnp
from jax import lax
from jax.experimental import pallas as pl
from jax.experimental.pallas import tpu as pltpu
from jax.experimental.pallas import tpu_sc as plsc

N = 10000
NPAD = 10240
E = 320000
EP = 331776
G = 64
NC = 2
NS = 16
ROWS_PER_TILE = NPAD // NS
K = 128
BN = 2048
NB = NPAD // BN


NBUF = 3


def _sc_pass(table, src_idx, dst_idx, zeros, width, feat_split, gather=True):
    if feat_split:
        edges_per_tile = EP // NS
    else:
        edges_per_tile = EP // (NC * NS)
    nchunks = edges_per_tile // K
    nsteps = -(-(nchunks + 2) // NBUF)

    mesh = plsc.VectorSubcoreMesh(core_axis_name="c", subcore_axis_name="s")

    @functools.partial(
        pl.kernel,
        out_type=jax.ShapeDtypeStruct((NC * NPAD, width), jnp.float32),
        mesh=mesh,
        scratch_types=[
            pltpu.VMEM((NBUF, K), jnp.int32),
            pltpu.VMEM((NBUF, K), jnp.int32),
            pltpu.VMEM((NBUF, K, width), jnp.float32),
            pltpu.VMEM_SHARED((NPAD, width), jnp.float32),
            pltpu.SemaphoreType.DMA((NBUF,)),
            pltpu.SemaphoreType.DMA((NBUF,)),
            pltpu.SemaphoreType.DMA((NBUF,)),
        ],
        compiler_params=pltpu.CompilerParams(use_tc_tiling_on_sc=False),
    )
    def k(table_h, src_h, dst_h, zero_h, out_h, src_v, dst_v, rows_v, agg_sh,
          sem_i, sem_g, sem_s):
        c = lax.axis_index("c")
        s = lax.axis_index("s")
        r0 = s * ROWS_PER_TILE
        pltpu.sync_copy(zero_h.at[pl.ds(r0, ROWS_PER_TILE)],
                        agg_sh.at[pl.ds(r0, ROWS_PER_TILE)])
        if not gather:
            pltpu.sync_copy(table_h.at[pl.ds(0, K)], rows_v.at[0])
        plsc.subcore_barrier()

        if feat_split:
            sbase = c * EP + s * edges_per_tile
            dbase = s * edges_per_tile
        else:
            sbase = (c * NS + s) * edges_per_tile
            dbase = sbase

        def idx_copies(g, b):
            out = [pltpu.make_async_copy(
                dst_h.at[pl.ds(dbase + g * K, K)], dst_v.at[b], sem_i.at[b])]
            if gather:
                out.append(pltpu.make_async_copy(
                    src_h.at[pl.ds(sbase + g * K, K)], src_v.at[b],
                    sem_i.at[b]))
            return out

        def gather_desc(b):
            return pltpu.make_async_copy(
                table_h.at[src_v.at[b]], rows_v.at[b], sem_g.at[b])

        def scatter_desc(b):
            rb = b if gather else 0
            return pltpu.make_async_copy(
                rows_v.at[rb], agg_sh.at[dst_v.at[b]], sem_s.at[b])

        for d in idx_copies(0, 0):
            d.start()

        def step(t, carry):
            for b in range(NBUF):
                g = t * NBUF + b
                bm1 = (b - 1) % NBUF
                bp1 = (b + 1) % NBUF

                @pl.when(g < nchunks)
                def _a():
                    for d in idx_copies(g, b):
                        d.wait()
                    if gather:
                        gather_desc(b).start()
                    else:
                        scatter_desc(b).start(add=True)

                if gather:
                    @pl.when((g >= 1) & (g <= nchunks))
                    def _b():
                        gather_desc(bm1).wait()
                        scatter_desc(bm1).start(add=True)

                @pl.when((g >= 2) & (g <= nchunks + 1))
                def _c():
                    scatter_desc(bp1).wait()

                @pl.when(g + 1 < nchunks)
                def _d():
                    for d in idx_copies(g + 1, bp1):
                        d.start()
            return carry

        lax.fori_loop(0, nsteps, step, 0)
        plsc.subcore_barrier()
        pltpu.sync_copy(agg_sh.at[pl.ds(r0, ROWS_PER_TILE)],
                        out_h.at[pl.ds(c * NPAD + r0, ROWS_PER_TILE)])

    return k(table, src_idx, dst_idx, zeros)


def _t1(xp, W1, cnt0, cnt1):
    def body(x_ref, w_ref, c0_ref, c1_ref, h_ref, dinv_ref):
        deg = c0_ref[...][:, :1] + c1_ref[...][:, :1]
        dinv = jnp.where(deg > 0.0, lax.rsqrt(deg), 0.0)
        dinv_ref[...] = dinv
        h = jnp.dot(x_ref[...], w_ref[...],
                    preferred_element_type=jnp.float32) * dinv
        h_ref[0] = h[:, :64]
        h_ref[1] = h[:, 64:]

    return pl.pallas_call(
        body,
        grid=(NB,),
        in_specs=[
            pl.BlockSpec((BN, 128), lambda i: (i, 0)),
            pl.BlockSpec((128, 128), lambda i: (0, 0)),
            pl.BlockSpec((BN, 16), lambda i: (i, 0)),
            pl.BlockSpec((BN, 16), lambda i: (i, 0)),
        ],
        out_specs=[
            pl.BlockSpec((NC, BN, 64), lambda i: (0, i, 0)),
            pl.BlockSpec((BN, 1), lambda i: (i, 0)),
        ],
        out_shape=[
            jax.ShapeDtypeStruct((NC, NPAD, 64), jnp.float32),
            jax.ShapeDtypeStruct((NPAD, 1), jnp.float32),
        ],
    )(xp, W1, cnt0, cnt1)


def _mid(a0, a1, dinv, b, W, split_out):
    outw = W.shape[1]

    def body(a0_ref, a1_ref, dinv_ref, b_ref, w_ref, out_ref):
        agg = jnp.concatenate([a0_ref[...], a1_ref[...]], axis=1)
        dinv = dinv_ref[...]
        z = jnp.maximum(agg * dinv + b_ref[...], 0.0)
        r = jnp.dot(z, w_ref[...], preferred_element_type=jnp.float32) * dinv
        if split_out:
            out_ref[0] = r[:, :64]
            out_ref[1] = r[:, 64:]
        else:
            out_ref[...] = r

    if split_out:
        out_spec = pl.BlockSpec((NC, BN, 64), lambda i: (0, i, 0))
        out_shape = jax.ShapeDtypeStruct((NC, NPAD, 64), jnp.float32)
    else:
        out_spec = pl.BlockSpec((BN, outw), lambda i: (i, 0))
        out_shape = jax.ShapeDtypeStruct((NPAD, outw), jnp.float32)

    return pl.pallas_call(
        body,
        grid=(NB,),
        in_specs=[
            pl.BlockSpec((BN, 64), lambda i: (i, 0)),
            pl.BlockSpec((BN, 64), lambda i: (i, 0)),
            pl.BlockSpec((BN, 1), lambda i: (i, 0)),
            pl.BlockSpec((1, 128), lambda i: (0, 0)),
            pl.BlockSpec((128, outw), lambda i: (0, 0)),
        ],
        out_specs=out_spec,
        out_shape=out_shape,
    )(a0, a1, dinv, b, W)


def _t4(g0, g1, dinv, b3p, batr):
    def body(g0_ref, g1_ref, dinv_ref, b_ref, bat_ref, out_ref):
        i = pl.program_id(0)

        @pl.when(i == 0)
        def _init():
            out_ref[...] = jnp.zeros_like(out_ref)

        p = (g0_ref[...] + g1_ref[...]) * dinv_ref[...] + b_ref[...]
        col = lax.broadcasted_iota(jnp.int32, (BN, 16), 1)
        p_aug = jnp.where(col == 15, 1.0, p)
        gids = lax.broadcasted_iota(jnp.int32, (G, BN), 0)
        mask = (bat_ref[...] == gids).astype(jnp.float32)
        out_ref[...] += jnp.dot(mask, p_aug,
                                preferred_element_type=jnp.float32)

        @pl.when(i == NB - 1)
        def _final():
            sums = out_ref[...]
            cnt = jnp.maximum(sums[:, 15:16], 1.0)
            m = sums / cnt
            ccol = lax.broadcasted_iota(jnp.int32, (G, 16), 1)
            logits = jnp.where(ccol < 10, m, -1e30)
            zz = logits - jnp.max(logits, axis=1, keepdims=True)
            ez = jnp.exp(zz)
            out_ref[...] = ez / jnp.sum(ez, axis=1, keepdims=True)

    return pl.pallas_call(
        body,
        grid=(NB,),
        in_specs=[
            pl.BlockSpec((BN, 16), lambda i: (i, 0)),
            pl.BlockSpec((BN, 16), lambda i: (i, 0)),
            pl.BlockSpec((BN, 1), lambda i: (i, 0)),
            pl.BlockSpec((1, 16), lambda i: (0, 0)),
            pl.BlockSpec((1, BN), lambda i: (0, i)),
        ],
        out_specs=pl.BlockSpec((G, 16), lambda i: (0, 0)),
        out_shape=jax.ShapeDtypeStruct((G, 16), jnp.float32),
    )(g0, g1, dinv, b3p, batr)


def kernel(x, edge_index, batch, W1, b1, W2, b2, W3, b3):
    src = edge_index[0]
    dst = edge_index[1]
    loop = jnp.arange(N, dtype=jnp.int32)
    padn = jnp.full((EP - E - N,), N, dtype=jnp.int32)
    srcf = jnp.concatenate([src, loop, padn])
    dstf = jnp.concatenate([dst, loop, padn])
    src2 = jnp.concatenate([srcf, srcf + NPAD])

    xp = jnp.zeros((NPAD, 128), jnp.float32).at[:N].set(x)
    batr = jnp.full((NPAD,), G, jnp.int32).at[:N].set(batch).reshape(1, NPAD)
    W3p = jnp.zeros((128, 16), jnp.float32).at[:, :10].set(W3)
    b1r = b1.reshape(1, 128)
    b2r = b2.reshape(1, 128)
    b3p = jnp.zeros((1, 16), jnp.float32).at[0, :10].set(b3)
    ones16 = jnp.ones((NPAD, 16), jnp.float32)
    z16 = jnp.zeros((NPAD, 16), jnp.float32)
    z64 = jnp.zeros((NPAD, 64), jnp.float32)

    cnt = _sc_pass(ones16, srcf, dstf, z16, 16, False)
    h1, dinv = _t1(xp, W1, cnt[:NPAD], cnt[NPAD:])
    agg1 = _sc_pass(h1.reshape(NC * NPAD, 64), src2, dstf, z64, 64, True)
    h2 = _mid(agg1[:NPAD], agg1[NPAD:], dinv, b1r, W2, True)
    agg2 = _sc_pass(h2.reshape(NC * NPAD, 64), src2, dstf, z64, 64, True)
    h3 = _mid(agg2[:NPAD], agg2[NPAD:], dinv, b2r, W3p, False)
    agg3 = _sc_pass(h3, srcf, dstf, z16, 16, False)
    out = _t4(agg3[:NPAD], agg3[NPAD:], dinv, b3p, batr)
    return out[:, :10]


# speedup vs baseline: 19.7489x; 1.0306x over previous
import functools

import jax
import jax.numpy as jnp
from jax import lax
from jax.experimental import pallas as pl
from jax.experimental.pallas import tpu as pltpu
from jax.experimental.pallas import tpu_sc as plsc

N = 10000
NPAD = 10240
E = 320000
EP = 331776
G = 64
NC = 2
NS = 16
ROWS_PER_TILE = NPAD // NS
K = 128
BN = 2048
NB = NPAD // BN
NBUF = 3


def _sc_pass(table, src_idx, dst_idx, zeros, width, gather=True):
    edges_per_tile = EP // (NC * NS)
    nchunks = edges_per_tile // K
    PERIOD = 6
    nsteps = -(-(nchunks + 2) // PERIOD)

    mesh = plsc.VectorSubcoreMesh(core_axis_name="c", subcore_axis_name="s")

    @functools.partial(
        pl.kernel,
        out_type=jax.ShapeDtypeStruct((NC * NPAD, width), jnp.float32),
        mesh=mesh,
        scratch_types=[
            pltpu.VMEM((3, K), jnp.int32),
            pltpu.VMEM((3, K), jnp.int32),
            pltpu.VMEM((2, K, width), jnp.float32),
            pltpu.VMEM_SHARED((NPAD, width), jnp.float32),
            pltpu.SemaphoreType.DMA((3,)),
            pltpu.SemaphoreType.DMA((2,)),
            pltpu.SemaphoreType.DMA((2,)),
        ],
        compiler_params=pltpu.CompilerParams(use_tc_tiling_on_sc=False),
    )
    def k(table_h, src_h, dst_h, zero_h, out_h, src_v, dst_v, rows_v, agg_sh,
          sem_i, sem_g, sem_s):
        c = lax.axis_index("c")
        s = lax.axis_index("s")
        r0 = s * ROWS_PER_TILE
        for j in range(ROWS_PER_TILE // K):
            pltpu.sync_copy(zero_h, agg_sh.at[pl.ds(r0 + j * K, K)])
        if not gather:
            pltpu.sync_copy(table_h.at[pl.ds(0, K)], rows_v.at[0])
        plsc.subcore_barrier()

        ebase = (c * NS + s) * edges_per_tile

        def idx_copies(g, ib):
            out = [pltpu.make_async_copy(
                dst_h.at[pl.ds(ebase + g * K, K)], dst_v.at[ib],
                sem_i.at[ib])]
            if gather:
                out.append(pltpu.make_async_copy(
                    src_h.at[pl.ds(ebase + g * K, K)], src_v.at[ib],
                    sem_i.at[ib]))
            return out

        def gather_desc(b, ib):
            return pltpu.make_async_copy(
                table_h.at[src_v.at[ib]], rows_v.at[b], sem_g.at[b])

        def scatter_desc(b, ib):
            rb = b if gather else 0
            return pltpu.make_async_copy(
                rows_v.at[rb], agg_sh.at[dst_v.at[ib]], sem_s.at[b])

        for d in idx_copies(0, 0):
            d.start()

        def step(t, carry):
            for u in range(PERIOD):
                g = t * PERIOD + u
                b = u % 2
                ib = u % 3

                @pl.when((g >= 2) & (g <= nchunks + 1))
                def _c():
                    scatter_desc(b, (u - 2) % 3).wait()

                @pl.when(g < nchunks)
                def _a():
                    for d in idx_copies(g, ib):
                        d.wait()
                    if gather:
                        gather_desc(b, ib).start()
                    else:
                        scatter_desc(b, ib).start(add=True)

                if gather:
                    @pl.when((g >= 1) & (g <= nchunks))
                    def _b():
                        gather_desc(1 - b, (u - 1) % 3).wait()
                        scatter_desc(1 - b, (u - 1) % 3).start(add=True)

                @pl.when(g + 1 < nchunks)
                def _d():
                    for d in idx_copies(g + 1, (u + 1) % 3):
                        d.start()
            return carry

        lax.fori_loop(0, nsteps, step, 0)
        plsc.subcore_barrier()
        pltpu.sync_copy(agg_sh.at[pl.ds(r0, ROWS_PER_TILE)],
                        out_h.at[pl.ds(c * NPAD + r0, ROWS_PER_TILE)])

    return k(table, src_idx, dst_idx, zeros)


def _t1(xp, W1, cnt0, cnt1):
    def body(x_ref, w_ref, c0_ref, c1_ref, h_ref, dinv_ref):
        deg = c0_ref[...][:, :1] + c1_ref[...][:, :1]
        dinv = jnp.where(deg > 0.0, lax.rsqrt(deg), 0.0)
        dinv_ref[...] = dinv
        h_ref[...] = jnp.dot(x_ref[...], w_ref[...],
                             preferred_element_type=jnp.float32) * dinv

    return pl.pallas_call(
        body,
        grid=(NB,),
        in_specs=[
            pl.BlockSpec((BN, 128), lambda i: (i, 0)),
            pl.BlockSpec((128, 128), lambda i: (0, 0)),
            pl.BlockSpec((BN, 16), lambda i: (i, 0)),
            pl.BlockSpec((BN, 16), lambda i: (i, 0)),
        ],
        out_specs=[
            pl.BlockSpec((BN, 128), lambda i: (i, 0)),
            pl.BlockSpec((BN, 1), lambda i: (i, 0)),
        ],
        out_shape=[
            jax.ShapeDtypeStruct((NPAD, 128), jnp.float32),
            jax.ShapeDtypeStruct((NPAD, 1), jnp.float32),
        ],
    )(xp, W1, cnt0, cnt1)


def _mid(a0, a1, dinv, b, W):
    outw = W.shape[1]

    def body(a0_ref, a1_ref, dinv_ref, b_ref, w_ref, out_ref):
        agg = a0_ref[...] + a1_ref[...]
        dinv = dinv_ref[...]
        z = jnp.maximum(agg * dinv + b_ref[...], 0.0)
        out_ref[...] = jnp.dot(z, w_ref[...],
                               preferred_element_type=jnp.float32) * dinv

    return pl.pallas_call(
        body,
        grid=(NB,),
        in_specs=[
            pl.BlockSpec((BN, 128), lambda i: (i, 0)),
            pl.BlockSpec((BN, 128), lambda i: (i, 0)),
            pl.BlockSpec((BN, 1), lambda i: (i, 0)),
            pl.BlockSpec((1, 128), lambda i: (0, 0)),
            pl.BlockSpec((128, outw), lambda i: (0, 0)),
        ],
        out_specs=pl.BlockSpec((BN, outw), lambda i: (i, 0)),
        out_shape=jax.ShapeDtypeStruct((NPAD, outw), jnp.float32),
    )(a0, a1, dinv, b, W)


def _t4(g0, g1, dinv, b3p, batr):
    def body(g0_ref, g1_ref, dinv_ref, b_ref, bat_ref, out_ref):
        i = pl.program_id(0)

        @pl.when(i == 0)
        def _init():
            out_ref[...] = jnp.zeros_like(out_ref)

        p = (g0_ref[...] + g1_ref[...]) * dinv_ref[...] + b_ref[...]
        col = lax.broadcasted_iota(jnp.int32, (BN, 16), 1)
        p_aug = jnp.where(col == 15, 1.0, p)
        gids = lax.broadcasted_iota(jnp.int32, (G, BN), 0)
        mask = (bat_ref[...] == gids).astype(jnp.float32)
        out_ref[...] += jnp.dot(mask, p_aug,
                                preferred_element_type=jnp.float32)

        @pl.when(i == NB - 1)
        def _final():
            sums = out_ref[...]
            cnt = jnp.maximum(sums[:, 15:16], 1.0)
            m = sums / cnt
            ccol = lax.broadcasted_iota(jnp.int32, (G, 16), 1)
            logits = jnp.where(ccol < 10, m, -1e30)
            zz = logits - jnp.max(logits, axis=1, keepdims=True)
            ez = jnp.exp(zz)
            out_ref[...] = ez / jnp.sum(ez, axis=1, keepdims=True)

    return pl.pallas_call(
        body,
        grid=(NB,),
        in_specs=[
            pl.BlockSpec((BN, 16), lambda i: (i, 0)),
            pl.BlockSpec((BN, 16), lambda i: (i, 0)),
            pl.BlockSpec((BN, 1), lambda i: (i, 0)),
            pl.BlockSpec((1, 16), lambda i: (0, 0)),
            pl.BlockSpec((1, BN), lambda i: (0, i)),
        ],
        out_specs=pl.BlockSpec((G, 16), lambda i: (0, 0)),
        out_shape=jax.ShapeDtypeStruct((G, 16), jnp.float32),
    )(g0, g1, dinv, b3p, batr)


def kernel(x, edge_index, batch, W1, b1, W2, b2, W3, b3):
    src = edge_index[0]
    dst = edge_index[1]
    loop = jnp.arange(N, dtype=jnp.int32)
    padn = jnp.full((EP - E - N,), N, dtype=jnp.int32)
    srcf = jnp.concatenate([src, loop, padn])
    dstf = jnp.concatenate([dst, loop, padn])

    xp = jnp.zeros((NPAD, 128), jnp.float32).at[:N].set(x)
    batr = jnp.full((NPAD,), G, jnp.int32).at[:N].set(batch).reshape(1, NPAD)
    W3p = jnp.zeros((128, 16), jnp.float32).at[:, :10].set(W3)
    b1r = b1.reshape(1, 128)
    b2r = b2.reshape(1, 128)
    b3p = jnp.zeros((1, 16), jnp.float32).at[0, :10].set(b3)
    ones16 = jnp.ones((NPAD, 16), jnp.float32)
    z16 = jnp.zeros((K, 16), jnp.float32)
    z128 = jnp.zeros((K, 128), jnp.float32)

    cnt = _sc_pass(ones16, srcf, dstf, z16, 16, gather=False)
    h1, dinv = _t1(xp, W1, cnt[:NPAD], cnt[NPAD:])
    agg1 = _sc_pass(h1, srcf, dstf, z128, 128)
    h2 = _mid(agg1[:NPAD], agg1[NPAD:], dinv, b1r, W2)
    agg2 = _sc_pass(h2, srcf, dstf, z128, 128)
    h3 = _mid(agg2[:NPAD], agg2[NPAD:], dinv, b2r, W3p)
    agg3 = _sc_pass(h3, srcf, dstf, z16, 16)
    out = _t4(agg3[:NPAD], agg3[NPAD:], dinv, b3p, batr)
    return out[:, :10]
